# baseline (speedup 1.0000x reference)
import functools

import jax
import jax.numpy as jnp
from jax import lax
from jax.experimental import pallas as pl
from jax.experimental.pallas import tpu as pltpu
from jax.experimental.pallas import tpu_sc as plsc

USER_LEN = 1000000
MOVIE_LEN = 100000
EMBED = 32
BATCH = 16384

RRELU_SLOPE = (1.0 / 8.0 + 1.0 / 3.0) / 2.0

_INFO = plsc.get_sparse_core_info()
_NC = _INFO.num_cores
_NS = _INFO.num_subcores
_NW = _NC * _NS
_B_PER_W = BATCH // _NW
_CHUNK = 128
_NCHUNK = _B_PER_W // _CHUNK
_TC = 128


def _rrelu(x):
    return jnp.where(x >= 0, x, x * RRELU_SLOPE)


_QW = 512


def _make_detile(n_rows):
    n_pad = ((n_rows + _TC - 1) // _TC) * _TC
    n_cols = (n_pad + _QW - 1) // _QW
    per_w = (n_cols + _NW - 1) // _NW

    def body(wt_hbm, flat_hbm, blk_v, out_v, sem_in, sem_out):
        wid = lax.axis_index("s") * _NC + lax.axis_index("c")
        lo = wid * per_w
        hi = jnp.minimum(lo + per_w, n_cols)
        lane = lax.iota(jnp.int32, 16)

        def off_of(tc):
            return pl.multiple_of(
                jnp.minimum(tc * _QW, n_pad - _QW), _TC)

        def copy_in(tc, slot):
            return pltpu.make_async_copy(
                wt_hbm.at[:, pl.ds(off_of(tc), _QW)],
                blk_v.at[slot], sem_in)

        def copy_out(tc, slot):
            return pltpu.make_async_copy(
                out_v.at[pl.ds(slot * (_QW * EMBED), _QW * EMBED)],
                flat_hbm.at[pl.ds(off_of(tc) * EMBED, _QW * EMBED)], sem_out)

        n_in = 4
        n_out = 2

        @pl.when(lo < hi)
        def _():
            def prime(tc, carry):
                copy_in(tc, lax.rem(tc - lo, n_in)).start()
                return carry

            lax.fori_loop(lo, jnp.minimum(lo + n_in, hi), prime, 0)

            def step(tc, carry):
                par = lax.rem(tc - lo, n_in)
                opar = lax.rem(tc - lo, n_out)
                copy_in(tc, par).wait()

                @pl.when(tc - n_out >= lo)
                def _():
                    copy_out(tc - n_out, opar).wait()

                obase = opar * (_QW * EMBED)
                evecs = [lane * EMBED + ((e + lane) & 15) + (e & 16)
                         for e in range(EMBED)]
                for e in range(EMBED):
                    for c0 in range(0, _QW, 16):
                        v = blk_v[par, e, pl.ds(c0, 16)]
                        sidx = (obase + c0 * EMBED) + evecs[e]
                        plsc.store_scatter(out_v, [sidx], v)
                copy_out(tc, opar).start()

                @pl.when(tc + n_in < hi)
                def _():
                    copy_in(tc + n_in, par).start()
                return carry

            lax.fori_loop(lo, hi, step, 0)

            def drain(tc, carry):
                copy_out(tc, lax.rem(tc - lo, n_out)).wait()
                return carry

            lax.fori_loop(jnp.maximum(lo, hi - n_out), hi, drain, 0)

    return functools.partial(
        pl.kernel,
        out_type=jax.ShapeDtypeStruct((n_pad * EMBED,), jnp.float32),
        mesh=plsc.VectorSubcoreMesh(
            core_axis_name="c", subcore_axis_name="s"),
        compiler_params=pltpu.CompilerParams(needs_layout_passes=False),
        scratch_types=[
            pltpu.VMEM((4, EMBED, _QW), jnp.float32),
            pltpu.VMEM((2 * _QW * EMBED,), jnp.float32),
            pltpu.SemaphoreType.DMA,
            pltpu.SemaphoreType.DMA,
        ],
    )(lambda wt, flat, *s: body(wt, flat, *s)), n_pad


_detile_w0, _W0_PAD = _make_detile(USER_LEN + 1)
_detile_w1, _W1_PAD = _make_detile(MOVIE_LEN + 1)


def _sc_lookup(seq0f_hbm, seq1f_hbm,
               w0_hbm, w1_hbm, b0_hbm, b1_hbm,
               out_hbm,
               idx0f_v, idx1f_v,
               rows0_v, rows1_v, bias0_v, bias1_v,
               out_v, sem):
    wid = lax.axis_index("s") * _NC + lax.axis_index("c")

    base_f = wid * _B_PER_W
    pltpu.sync_copy(seq0f_hbm.at[pl.ds(base_f, _B_PER_W)], idx0f_v)
    pltpu.sync_copy(seq1f_hbm.at[pl.ds(base_f, _B_PER_W)], idx1f_v)

    copies = []
    for j in range(_NCHUNK):
        dst = pl.ds(j * _CHUNK, _CHUNK)
        isl = pl.ds(j * _CHUNK, _CHUNK)
        copies.append(pltpu.async_copy(
            w0_hbm.at[idx0f_v.at[isl]], rows0_v.at[dst], sem))
        copies.append(pltpu.async_copy(
            w1_hbm.at[idx1f_v.at[isl]], rows1_v.at[dst], sem))
        copies.append(pltpu.async_copy(
            b0_hbm.at[idx0f_v.at[isl]], bias0_v.at[dst], sem))
        copies.append(pltpu.async_copy(
            b1_hbm.at[idx1f_v.at[isl]], bias1_v.at[dst], sem))
    for c in copies:
        c.wait()

    lane = lax.iota(jnp.int32, 16)

    def body(g, carry):
        base = g * 16
        item = base + lane
        rot0 = idx0f_v[pl.ds(base, 16)] & 15
        rot1 = idx1f_v[pl.ds(base, 16)] & 15
        acc = bias0_v[pl.ds(base, 16)] + bias1_v[pl.ds(base, 16)]
        for e in range(EMBED):
            hi = e & 16
            e0 = hi + ((e + rot0) & 15)
            e1 = hi + ((e + rot1) & 15)
            g0 = plsc.load_gather(rows0_v, [item, e0])
            g1 = plsc.load_gather(rows1_v, [item, e1])
            acc = acc + _rrelu(g0) * _rrelu(g1)
        out_v[pl.ds(base, 16)] = acc
        return carry

    lax.fori_loop(0, _B_PER_W // 16, body, 0)

    pltpu.sync_copy(out_v, out_hbm.at[pl.ds(wid * _B_PER_W, _B_PER_W)])


@functools.partial(
    pl.kernel,
    out_type=jax.ShapeDtypeStruct((BATCH,), jnp.float32),
    mesh=plsc.VectorSubcoreMesh(core_axis_name="c", subcore_axis_name="s"),
    compiler_params=pltpu.CompilerParams(
        needs_layout_passes=False, use_tc_tiling_on_sc=False),
    scratch_types=[
        pltpu.VMEM((_B_PER_W,), jnp.int32),
        pltpu.VMEM((_B_PER_W,), jnp.int32),
        pltpu.VMEM((_B_PER_W, EMBED), jnp.float32),
        pltpu.VMEM((_B_PER_W, EMBED), jnp.float32),
        pltpu.VMEM((_B_PER_W,), jnp.float32),
        pltpu.VMEM((_B_PER_W,), jnp.float32),
        pltpu.VMEM((_B_PER_W,), jnp.float32),
        pltpu.SemaphoreType.DMA,
    ],
)
def _mf_forward(seq0f, seq1f, w0, w1, b0, b1, out, *scratch):
    _sc_lookup(seq0f, seq1f, w0, w1, b0, b1, out, *scratch)


def kernel(seq0, seq1, W0, W1, B0, B1):
    seq0f = seq0.astype(jnp.int32)
    seq1f = seq1.astype(jnp.int32)
    w0_flat = _detile_w0(W0.T)
    w1_flat = _detile_w1(W1.T)
    w0 = w0_flat.reshape(_W0_PAD, EMBED)
    w1 = w1_flat.reshape(_W1_PAD, EMBED)
    out = _mf_forward(seq0f, seq1f, w0, w1,
                      B0.reshape(-1), B1.reshape(-1))
    return out.reshape(BATCH, 1)


# speedup vs baseline: 1.0377x; 1.0377x over previous
import functools

import jax
import jax.numpy as jnp
from jax import lax
from jax.experimental import pallas as pl
from jax.experimental.pallas import tpu as pltpu
from jax.experimental.pallas import tpu_sc as plsc

USER_LEN = 1000000
MOVIE_LEN = 100000
EMBED = 32
BATCH = 16384

RRELU_SLOPE = (1.0 / 8.0 + 1.0 / 3.0) / 2.0

_INFO = plsc.get_sparse_core_info()
_NC = _INFO.num_cores
_NS = _INFO.num_subcores
_NW = _NC * _NS
_B_PER_W = BATCH // _NW
_CHUNK = 128
_NCHUNK = _B_PER_W // _CHUNK
_TC = 128


def _rrelu(x):
    return jnp.where(x >= 0, x, x * RRELU_SLOPE)


_QW = 512


def _make_detile(n_rows):
    n_pad = ((n_rows + _TC - 1) // _TC) * _TC
    n_cols = (n_pad + _QW - 1) // _QW
    per_w = (n_cols + _NW - 1) // _NW

    def body(wt_hbm, flat_hbm, blk_v, out_v, sem_in, sem_out):
        wid = lax.axis_index("s") * _NC + lax.axis_index("c")
        lo = wid * per_w
        hi = jnp.minimum(lo + per_w, n_cols)
        lane = lax.iota(jnp.int32, 16)

        def off_of(tc):
            return pl.multiple_of(
                jnp.minimum(tc * _QW, n_pad - _QW), _TC)

        def copy_in(tc, slot):
            return pltpu.make_async_copy(
                wt_hbm.at[:, pl.ds(off_of(tc), _QW)],
                blk_v.at[slot], sem_in)

        def copy_out(tc, slot):
            return pltpu.make_async_copy(
                out_v.at[pl.ds(slot * (_QW * EMBED), _QW * EMBED)],
                flat_hbm.at[pl.ds(off_of(tc) * EMBED, _QW * EMBED)], sem_out)

        n_in = 4
        n_out = 2

        @pl.when(lo < hi)
        def _():
            def prime(tc, carry):
                copy_in(tc, lax.rem(tc - lo, n_in)).start()
                return carry

            lax.fori_loop(lo, jnp.minimum(lo + n_in, hi), prime, 0)

            def step(tc, carry):
                par = lax.rem(tc - lo, n_in)
                opar = lax.rem(tc - lo, n_out)
                copy_in(tc, par).wait()

                @pl.when(tc - n_out >= lo)
                def _():
                    copy_out(tc - n_out, opar).wait()

                obase = opar * (_QW * EMBED)
                evecs = [lane * EMBED + ((e + lane) & 15) + (e & 16)
                         for e in range(EMBED)]
                for c0 in range(0, _QW, 16):
                    bc = jnp.full((16,), obase + c0 * EMBED, jnp.int32)
                    for e in range(EMBED):
                        v = blk_v[par, e, pl.ds(c0, 16)]
                        plsc.store_scatter(out_v, [bc + evecs[e]], v)
                copy_out(tc, opar).start()

                @pl.when(tc + n_in < hi)
                def _():
                    copy_in(tc + n_in, par).start()
                return carry

            lax.fori_loop(lo, hi, step, 0)

            def drain(tc, carry):
                copy_out(tc, lax.rem(tc - lo, n_out)).wait()
                return carry

            lax.fori_loop(jnp.maximum(lo, hi - n_out), hi, drain, 0)

    return functools.partial(
        pl.kernel,
        out_type=jax.ShapeDtypeStruct((n_pad * EMBED,), jnp.float32),
        mesh=plsc.VectorSubcoreMesh(
            core_axis_name="c", subcore_axis_name="s"),
        compiler_params=pltpu.CompilerParams(needs_layout_passes=False),
        scratch_types=[
            pltpu.VMEM((4, EMBED, _QW), jnp.float32),
            pltpu.VMEM((2 * _QW * EMBED,), jnp.float32),
            pltpu.SemaphoreType.DMA,
            pltpu.SemaphoreType.DMA,
        ],
    )(lambda wt, flat, *s: body(wt, flat, *s)), n_pad


_detile_w0, _W0_PAD = _make_detile(USER_LEN + 1)
_detile_w1, _W1_PAD = _make_detile(MOVIE_LEN + 1)


def _sc_lookup(seq0f_hbm, seq1f_hbm,
               w0_hbm, w1_hbm, b0_hbm, b1_hbm,
               out_hbm,
               idx0f_v, idx1f_v,
               rows0_v, rows1_v, bias0_v, bias1_v,
               out_v, sem):
    wid = lax.axis_index("s") * _NC + lax.axis_index("c")

    base_f = wid * _B_PER_W
    pltpu.sync_copy(seq0f_hbm.at[pl.ds(base_f, _B_PER_W)], idx0f_v)
    pltpu.sync_copy(seq1f_hbm.at[pl.ds(base_f, _B_PER_W)], idx1f_v)

    copies = []
    for j in range(_NCHUNK):
        dst = pl.ds(j * _CHUNK, _CHUNK)
        isl = pl.ds(j * _CHUNK, _CHUNK)
        copies.append(pltpu.async_copy(
            w0_hbm.at[idx0f_v.at[isl]], rows0_v.at[dst], sem))
        copies.append(pltpu.async_copy(
            w1_hbm.at[idx1f_v.at[isl]], rows1_v.at[dst], sem))
        copies.append(pltpu.async_copy(
            b0_hbm.at[idx0f_v.at[isl]], bias0_v.at[dst], sem))
        copies.append(pltpu.async_copy(
            b1_hbm.at[idx1f_v.at[isl]], bias1_v.at[dst], sem))
    for c in copies:
        c.wait()

    lane = lax.iota(jnp.int32, 16)

    def body(g, carry):
        base = g * 16
        item = base + lane
        rot0 = idx0f_v[pl.ds(base, 16)] & 15
        rot1 = idx1f_v[pl.ds(base, 16)] & 15
        acc = bias0_v[pl.ds(base, 16)] + bias1_v[pl.ds(base, 16)]
        for e in range(EMBED):
            hi = e & 16
            e0 = hi + ((e + rot0) & 15)
            e1 = hi + ((e + rot1) & 15)
            g0 = plsc.load_gather(rows0_v, [item, e0])
            g1 = plsc.load_gather(rows1_v, [item, e1])
            acc = acc + _rrelu(g0) * _rrelu(g1)
        out_v[pl.ds(base, 16)] = acc
        return carry

    lax.fori_loop(0, _B_PER_W // 16, body, 0)

    pltpu.sync_copy(out_v, out_hbm.at[pl.ds(wid * _B_PER_W, _B_PER_W)])


@functools.partial(
    pl.kernel,
    out_type=jax.ShapeDtypeStruct((BATCH,), jnp.float32),
    mesh=plsc.VectorSubcoreMesh(core_axis_name="c", subcore_axis_name="s"),
    compiler_params=pltpu.CompilerParams(
        needs_layout_passes=False, use_tc_tiling_on_sc=False),
    scratch_types=[
        pltpu.VMEM((_B_PER_W,), jnp.int32),
        pltpu.VMEM((_B_PER_W,), jnp.int32),
        pltpu.VMEM((_B_PER_W, EMBED), jnp.float32),
        pltpu.VMEM((_B_PER_W, EMBED), jnp.float32),
        pltpu.VMEM((_B_PER_W,), jnp.float32),
        pltpu.VMEM((_B_PER_W,), jnp.float32),
        pltpu.VMEM((_B_PER_W,), jnp.float32),
        pltpu.SemaphoreType.DMA,
    ],
)
def _mf_forward(seq0f, seq1f, w0, w1, b0, b1, out, *scratch):
    _sc_lookup(seq0f, seq1f, w0, w1, b0, b1, out, *scratch)


def kernel(seq0, seq1, W0, W1, B0, B1):
    seq0f = seq0.astype(jnp.int32)
    seq1f = seq1.astype(jnp.int32)
    w0_flat = _detile_w0(W0.T)
    w1_flat = _detile_w1(W1.T)
    w0 = w0_flat.reshape(_W0_PAD, EMBED)
    w1 = w1_flat.reshape(_W1_PAD, EMBED)
    out = _mf_forward(seq0f, seq1f, w0, w1,
                      B0.reshape(-1), B1.reshape(-1))
    return out.reshape(BATCH, 1)
